# Initial kernel scaffold; baseline (speedup 1.0000x reference)
import functools

import jax
import jax.numpy as jnp
from jax import lax
from jax.experimental import pallas as pl
from jax.experimental.pallas import tpu as pltpu
from jax.experimental.pallas import tpu_sc as plsc

VOCAB = 1000000
D = 100
SEQ = 40
BATCH = 16384

NC = 2
NS = 16
NW = NC * NS

ROWS_PER_W = BATCH // NW
ROWS_PER_CHUNK = 2
IDX_PER_CHUNK = ROWS_PER_CHUNK * SEQ
CHUNKS = ROWS_PER_W // ROWS_PER_CHUNK

OFFS = (0, 16, 32, 48, 64, 80, 84)
INV_SEQ = 1.0 / SEQ


def _body(idx_hbm, table_hbm, out_hbm, idx_v, buf0, buf1, out_v, sem0, sem1):
    w = lax.axis_index("s") * NC + lax.axis_index("c")

    pltpu.sync_copy(idx_hbm.at[w], idx_v)

    bufs = (buf0, buf1)
    sems = (sem0, sem1)

    def start(c, b):
        pltpu.async_copy(table_hbm.at[idx_v.at[c]], bufs[b], sems[b])

    def wait(c, b):
        pltpu.make_async_copy(table_hbm.at[idx_v.at[c]], bufs[b], sems[b]).wait()

    def compute(c, b):
        buf = bufs[b]
        for r in range(ROWS_PER_CHUNK):
            def acc_step(s, acc):
                row = r * SEQ + s
                return tuple(a + buf[row, pl.ds(off, 16)]
                             for a, off in zip(acc, OFFS))

            zeros = tuple(jnp.zeros((16,), jnp.float32) for _ in OFFS)
            acc = lax.fori_loop(0, SEQ, acc_step, zeros)
            orow = c * ROWS_PER_CHUNK + r
            for a, off in zip(acc, OFFS):
                out_v[orow, pl.ds(off, 16)] = a * INV_SEQ

    start(0, 0)

    def loop_body(i, carry):
        g = i * 2
        start(g + 1, 1)
        wait(g, 0)
        compute(g, 0)

        @pl.when(g + 2 < CHUNKS)
        def _():
            start(g + 2, 0)

        wait(g + 1, 1)
        compute(g + 1, 1)
        return carry

    lax.fori_loop(0, CHUNKS // 2, loop_body, 0)

    pltpu.sync_copy(out_v, out_hbm.at[pl.ds(w * ROWS_PER_W, ROWS_PER_W)])


@jax.jit
def _swem(x_blocks, table):
    mesh = plsc.VectorSubcoreMesh(core_axis_name="c", subcore_axis_name="s")
    fn = functools.partial(
        pl.kernel,
        out_type=jax.ShapeDtypeStruct((BATCH, D), jnp.float32),
        mesh=mesh,
        scratch_types=[
            pltpu.VMEM((CHUNKS, IDX_PER_CHUNK), jnp.int32),
            pltpu.VMEM((IDX_PER_CHUNK, D), jnp.float32),
            pltpu.VMEM((IDX_PER_CHUNK, D), jnp.float32),
            pltpu.VMEM((ROWS_PER_W, D), jnp.float32),
            pltpu.SemaphoreType.DMA,
            pltpu.SemaphoreType.DMA,
        ],
    )(_body)
    return fn(x_blocks, table)


def kernel(x, lengths, table):
    del lengths
    x_blocks = x.astype(jnp.int32).reshape(NW, CHUNKS, IDX_PER_CHUNK)
    return _swem(x_blocks, table)


# speedup vs baseline: 1.0464x; 1.0464x over previous
import functools

import jax
import jax.numpy as jnp
from jax import lax
from jax.experimental import pallas as pl
from jax.experimental.pallas import tpu as pltpu
from jax.experimental.pallas import tpu_sc as plsc

VOCAB = 1000000
D = 100
DP = 128
SEQ = 40
BATCH = 16384

NC = 2
NS = 16
NW = NC * NS

ROWS_PER_W = BATCH // NW
ROWS_PER_CHUNK = 2
IDX_PER_CHUNK = ROWS_PER_CHUNK * SEQ
CHUNKS = ROWS_PER_W // ROWS_PER_CHUNK

OFFS = (0, 16, 32, 48, 64, 80, 84)
INV_SEQ = 1.0 / SEQ


def _body(idx_hbm, table_hbm, out_hbm, idx_v, buf0, buf1, out_v, sem0, sem1):
    w = lax.axis_index("s") * NC + lax.axis_index("c")

    pltpu.sync_copy(idx_hbm.at[w], idx_v)

    bufs = (buf0, buf1)
    sems = (sem0, sem1)

    def start(c, b):
        pltpu.async_copy(table_hbm.at[idx_v.at[c]], bufs[b], sems[b])

    def wait(c, b):
        pltpu.make_async_copy(table_hbm.at[idx_v.at[c]], bufs[b], sems[b]).wait()

    def compute(c, b):
        buf = bufs[b]
        for r in range(ROWS_PER_CHUNK):
            def acc_step(s, acc):
                row = r * SEQ + s
                return tuple(a + buf[row, pl.ds(off, 16)]
                             for a, off in zip(acc, OFFS))

            zeros = tuple(jnp.zeros((16,), jnp.float32) for _ in OFFS)
            acc = lax.fori_loop(0, SEQ, acc_step, zeros)
            orow = c * ROWS_PER_CHUNK + r
            for a, off in zip(acc, OFFS):
                out_v[orow, pl.ds(off, 16)] = a * INV_SEQ

    start(0, 0)

    def loop_body(i, carry):
        g = i * 2
        start(g + 1, 1)
        wait(g, 0)
        compute(g, 0)

        @pl.when(g + 2 < CHUNKS)
        def _():
            start(g + 2, 0)

        wait(g + 1, 1)
        compute(g + 1, 1)
        return carry

    lax.fori_loop(0, CHUNKS // 2, loop_body, 0)

    pltpu.sync_copy(out_v, out_hbm.at[pl.ds(w * ROWS_PER_W, ROWS_PER_W)])


@jax.jit
def _swem(x_blocks, table):
    table_p = jnp.pad(table, ((0, 0), (0, DP - D)))
    mesh = plsc.VectorSubcoreMesh(core_axis_name="c", subcore_axis_name="s")
    fn = functools.partial(
        pl.kernel,
        out_type=jax.ShapeDtypeStruct((BATCH, D), jnp.float32),
        mesh=mesh,
        scratch_types=[
            pltpu.VMEM((CHUNKS, IDX_PER_CHUNK), jnp.int32),
            pltpu.VMEM((IDX_PER_CHUNK, DP), jnp.float32),
            pltpu.VMEM((IDX_PER_CHUNK, DP), jnp.float32),
            pltpu.VMEM((ROWS_PER_W, D), jnp.float32),
            pltpu.SemaphoreType.DMA,
            pltpu.SemaphoreType.DMA,
        ],
    )(_body)
    return fn(x_blocks, table_p)


def kernel(x, lengths, table):
    del lengths
    x_blocks = x.astype(jnp.int32).reshape(NW, CHUNKS, IDX_PER_CHUNK)
    return _swem(x_blocks, table)


# speedup vs baseline: 2.3410x; 2.2373x over previous
import functools

import jax
import jax.numpy as jnp
from jax import lax
from jax.experimental import pallas as pl
from jax.experimental.pallas import tpu as pltpu
from jax.experimental.pallas import tpu_sc as plsc

VOCAB = 1000000
D = 100
DP = 128
SEQ = 40
BATCH = 16384

NC = 2
NS = 16
NW = NC * NS

ROWS_PER_W = BATCH // NW
ROWS_PER_CHUNK = 2
IDX_PER_CHUNK = ROWS_PER_CHUNK * SEQ
CHUNKS = ROWS_PER_W // ROWS_PER_CHUNK

OFFS = (0, 16, 32, 48, 64, 80, 84)
INV_SEQ = 1.0 / SEQ


def _body(idx_hbm, table_hbm, out_hbm, idx_v, buf0, buf1, out_v, sem0, sem1):
    w = lax.axis_index("s") * NC + lax.axis_index("c")

    pltpu.sync_copy(idx_hbm.at[w], idx_v)

    bufs = (buf0, buf1)
    sems = (sem0, sem1)

    def start(c, b):
        pltpu.async_copy(table_hbm.at[idx_v.at[c]], bufs[b], sems[b])

    def wait(c, b):
        pltpu.make_async_copy(table_hbm.at[idx_v.at[c]], bufs[b], sems[b]).wait()

    def compute(c, b):
        buf = bufs[b]
        for r in range(ROWS_PER_CHUNK):
            def acc_step(s, acc):
                row = r * SEQ + s
                return tuple(a + buf[row, pl.ds(off, 16)]
                             for a, off in zip(acc, OFFS))

            zeros = tuple(jnp.zeros((16,), jnp.float32) for _ in OFFS)
            acc = lax.fori_loop(0, SEQ, acc_step, zeros)
            orow = c * ROWS_PER_CHUNK + r
            for a, off in zip(acc, OFFS):
                out_v[orow, pl.ds(off, 16)] = a * INV_SEQ

    start(0, 0)

    def loop_body(i, carry):
        g = i * 2
        start(g + 1, 1)
        wait(g, 0)
        compute(g, 0)

        @pl.when(g + 2 < CHUNKS)
        def _():
            start(g + 2, 0)

        wait(g + 1, 1)
        compute(g + 1, 1)
        return carry

    lax.fori_loop(0, CHUNKS // 2, loop_body, 0)

    pltpu.sync_copy(out_v, out_hbm.at[pl.ds(w * ROWS_PER_W, ROWS_PER_W)])


PAD_ROWS = 10000


def _pad_body(t_ref, o_ref):
    o_ref[:, :D] = t_ref[...]


def _widen_table(table):
    return pl.pallas_call(
        _pad_body,
        out_shape=jax.ShapeDtypeStruct((VOCAB, DP), jnp.float32),
        grid=(VOCAB // PAD_ROWS,),
        in_specs=[pl.BlockSpec((PAD_ROWS, D), lambda i: (i, 0))],
        out_specs=pl.BlockSpec((PAD_ROWS, DP), lambda i: (i, 0)),
    )(table)


@jax.jit
def _swem(x_blocks, table):
    table_p = _widen_table(table)
    mesh = plsc.VectorSubcoreMesh(core_axis_name="c", subcore_axis_name="s")
    fn = functools.partial(
        pl.kernel,
        out_type=jax.ShapeDtypeStruct((BATCH, D), jnp.float32),
        mesh=mesh,
        scratch_types=[
            pltpu.VMEM((CHUNKS, IDX_PER_CHUNK), jnp.int32),
            pltpu.VMEM((IDX_PER_CHUNK, DP), jnp.float32),
            pltpu.VMEM((IDX_PER_CHUNK, DP), jnp.float32),
            pltpu.VMEM((ROWS_PER_W, D), jnp.float32),
            pltpu.SemaphoreType.DMA,
            pltpu.SemaphoreType.DMA,
        ],
    )(_body)
    return fn(x_blocks, table_p)


def kernel(x, lengths, table):
    del lengths
    x_blocks = x.astype(jnp.int32).reshape(NW, CHUNKS, IDX_PER_CHUNK)
    return _swem(x_blocks, table)
